# Initial kernel scaffold; baseline (speedup 1.0000x reference)
import functools

import jax
import jax.numpy as jnp
from jax import lax
from jax.experimental import pallas as pl
from jax.experimental.pallas import tpu as pltpu
from jax.experimental.pallas import tpu_sc as plsc

_N = 6400000
_G = 4096
_ROW = 128
_NROWS = _N // _ROW
_NC = 2
_NS = 16
_NW = _NC * _NS
_RPT = _NROWS // _NW
_EXTRA = _NROWS - _RPT * _NW
_PAGE = 71
_NPAGES = _RPT // _PAGE
_ZCHUNK = _G // _NS


def _sc_partials(src2, batch2):
    mesh = plsc.VectorSubcoreMesh(core_axis_name="c", subcore_axis_name="s")

    @functools.partial(
        pl.kernel,
        out_type=jax.ShapeDtypeStruct((_NC, _G), jnp.float32),
        mesh=mesh,
        scratch_types=[
            pltpu.VMEM((_PAGE, _ROW), jnp.float32),
            pltpu.VMEM((_PAGE, _ROW), jnp.int32),
            pltpu.VMEM((_ZCHUNK,), jnp.float32),
            pltpu.VMEM_SHARED((_G,), jnp.float32),
            pltpu.SemaphoreType.DMA,
        ],
    )
    def k(src_hbm, idx_hbm, out_hbm, sv, iv, zv, acc, sem):
        cid = lax.axis_index("c")
        sid = lax.axis_index("s")
        wid = cid * _NS + sid

        @pl.loop(0, _ZCHUNK // 16, unroll=8)
        def _(i):
            zv[pl.ds(i * 16, 16)] = jnp.zeros((16,), jnp.float32)

        pltpu.sync_copy(zv, acc.at[pl.ds(sid * _ZCHUNK, _ZCHUNK)])
        plsc.subcore_barrier()

        base = wid * _RPT

        @pl.loop(0, _NPAGES)
        def _(p):
            r0 = base + p * _PAGE
            pltpu.sync_copy(src_hbm.at[pl.ds(r0, _PAGE)], sv)
            pltpu.sync_copy(idx_hbm.at[pl.ds(r0, _PAGE)], iv)

            @pl.loop(0, _PAGE, unroll=8)
            def _(j):
                pltpu.async_copy(sv.at[j], acc.at[iv.at[j]], sem, add=True)

            pltpu.make_async_copy(src_hbm.at[pl.ds(r0, _PAGE)], sv, sem).wait()

        @pl.when(wid < _EXTRA)
        def _():
            r = _NW * _RPT + wid
            pltpu.sync_copy(src_hbm.at[pl.ds(r, 1)], sv.at[pl.ds(0, 1)])
            pltpu.sync_copy(idx_hbm.at[pl.ds(r, 1)], iv.at[pl.ds(0, 1)])
            pltpu.sync_copy(sv.at[0], acc.at[iv.at[0]], add=True)

        plsc.subcore_barrier()

        @pl.when(sid == 0)
        def _():
            pltpu.sync_copy(acc, out_hbm.at[cid])

    return k(src2, batch2)


def _combine(partials):
    def body(p_ref, o_ref):
        o_ref[...] = p_ref[0, :] + p_ref[1, :]

    return pl.pallas_call(
        body, out_shape=jax.ShapeDtypeStruct((_G,), jnp.float32)
    )(partials)


@jax.jit
def _run(src, batch):
    src2 = src.reshape(_NROWS, _ROW)
    batch2 = batch.reshape(_NROWS, _ROW)
    return _combine(_sc_partials(src2, batch2))


def kernel(src, batch, cell_volume):
    del cell_volume
    return _run(src, batch)


# speedup vs baseline: 13.5808x; 13.5808x over previous
import functools

import jax
import jax.numpy as jnp
from jax import lax
from jax.experimental import pallas as pl
from jax.experimental.pallas import tpu as pltpu
from jax.experimental.pallas import tpu_sc as plsc

_N = 6400000
_G = 4096
_ROW = 128
_NROWS = _N // _ROW
_NC = 2
_NS = 16
_NW = _NC * _NS
_RPT = 1560
_XROWS = 8
_XTILES = (_NROWS - _RPT * _NW) // _XROWS
_PAGE = 120
_NPAGES = _RPT // _PAGE
_ZCHUNK = _G // _NS


def _sc_partials(src2, batch2):
    mesh = plsc.VectorSubcoreMesh(core_axis_name="c", subcore_axis_name="s")

    @functools.partial(
        pl.kernel,
        out_type=jax.ShapeDtypeStruct((_NC, _G), jnp.float32),
        mesh=mesh,
        scratch_types=[
            pltpu.VMEM((_PAGE, _ROW), jnp.float32),
            pltpu.VMEM((_PAGE, _ROW), jnp.int32),
            pltpu.VMEM((_ZCHUNK,), jnp.float32),
            pltpu.VMEM_SHARED((_G,), jnp.float32),
            pltpu.SemaphoreType.DMA,
        ],
    )
    def k(src_hbm, idx_hbm, out_hbm, sv, iv, zv, acc, sem):
        cid = lax.axis_index("c")
        sid = lax.axis_index("s")
        wid = cid * _NS + sid

        @pl.loop(0, _ZCHUNK // 16, unroll=8)
        def _(i):
            zv[pl.ds(i * 16, 16)] = jnp.zeros((16,), jnp.float32)

        pltpu.sync_copy(zv, acc.at[pl.ds(sid * _ZCHUNK, _ZCHUNK)])
        plsc.subcore_barrier()

        base = wid * _RPT

        @pl.loop(0, _NPAGES)
        def _(p):
            r0 = base + p * _PAGE
            pltpu.sync_copy(src_hbm.at[pl.ds(r0, _PAGE)], sv)
            pltpu.sync_copy(idx_hbm.at[pl.ds(r0, _PAGE)], iv)

            @pl.loop(0, _PAGE, unroll=8)
            def _(j):
                pltpu.async_copy(sv.at[j], acc.at[iv.at[j]], sem, add=True)

            @pl.loop(0, _PAGE, unroll=8)
            def _(j):
                pltpu.make_async_copy(sv.at[j], acc.at[iv.at[j]], sem).wait()

        @pl.when(wid < _XTILES)
        def _():
            r = _NW * _RPT + wid * _XROWS
            pltpu.sync_copy(src_hbm.at[pl.ds(r, _XROWS)], sv.at[pl.ds(0, _XROWS)])
            pltpu.sync_copy(idx_hbm.at[pl.ds(r, _XROWS)], iv.at[pl.ds(0, _XROWS)])
            for j in range(_XROWS):
                pltpu.sync_copy(sv.at[j], acc.at[iv.at[j]], add=True)

        plsc.subcore_barrier()

        @pl.when(sid == 0)
        def _():
            pltpu.sync_copy(acc, out_hbm.at[cid])

    return k(src2, batch2)


def _combine(partials):
    def body(p_ref, o_ref):
        o_ref[...] = p_ref[0, :] + p_ref[1, :]

    return pl.pallas_call(
        body, out_shape=jax.ShapeDtypeStruct((_G,), jnp.float32)
    )(partials)


@jax.jit
def _run(src, batch):
    src2 = src.reshape(_NROWS, _ROW)
    batch2 = batch.reshape(_NROWS, _ROW)
    return _combine(_sc_partials(src2, batch2))


def kernel(src, batch, cell_volume):
    del cell_volume
    return _run(src, batch)


# speedup vs baseline: 14.4855x; 1.0666x over previous
import functools

import jax
import jax.numpy as jnp
from jax import lax
from jax.experimental import pallas as pl
from jax.experimental.pallas import tpu as pltpu
from jax.experimental.pallas import tpu_sc as plsc

_N = 6400000
_G = 4096
_NC = 2
_NS = 16
_NW = _NC * _NS
_EPT = 199680
_XELEM = 1024
_XTILES = (_N - _EPT * _NW) // _XELEM
_PAGE = 16640
_NPAGES = _EPT // _PAGE
_ZCHUNK = _G // _NS


def _sc_partials(src1, batch1):
    mesh = plsc.VectorSubcoreMesh(core_axis_name="c", subcore_axis_name="s")

    @functools.partial(
        pl.kernel,
        out_type=jax.ShapeDtypeStruct((_NC, _G), jnp.float32),
        mesh=mesh,
        scratch_types=[
            pltpu.VMEM((_PAGE,), jnp.float32),
            pltpu.VMEM((_PAGE,), jnp.int32),
            pltpu.VMEM((_PAGE,), jnp.float32),
            pltpu.VMEM((_PAGE,), jnp.int32),
            pltpu.VMEM((_XELEM,), jnp.float32),
            pltpu.VMEM((_XELEM,), jnp.int32),
            pltpu.VMEM((_ZCHUNK,), jnp.float32),
            pltpu.VMEM_SHARED((_G,), jnp.float32),
            pltpu.SemaphoreType.DMA,
            pltpu.SemaphoreType.DMA,
        ],
    )
    def k(src_hbm, idx_hbm, out_hbm, sv0, iv0, sv1, iv1, xv, xi, zv, acc,
          sem_in, sem_sc):
        cid = lax.axis_index("c")
        sid = lax.axis_index("s")
        wid = cid * _NS + sid

        @pl.loop(0, _ZCHUNK // 16, unroll=8)
        def _(i):
            zv[pl.ds(i * 16, 16)] = jnp.zeros((16,), jnp.float32)

        pltpu.sync_copy(zv, acc.at[pl.ds(sid * _ZCHUNK, _ZCHUNK)])
        plsc.subcore_barrier()

        base = wid * _EPT

        def start_in(p, sv_, iv_):
            e0 = base + p * _PAGE
            pltpu.async_copy(src_hbm.at[pl.ds(e0, _PAGE)], sv_, sem_in)
            pltpu.async_copy(idx_hbm.at[pl.ds(e0, _PAGE)], iv_, sem_in)

        def wait_in(p, sv_, iv_):
            e0 = base + p * _PAGE
            pltpu.make_async_copy(
                src_hbm.at[pl.ds(e0, _PAGE)], sv_, sem_in).wait()
            pltpu.make_async_copy(
                idx_hbm.at[pl.ds(e0, _PAGE)], iv_, sem_in).wait()

        def fire(sv_, iv_):
            pltpu.async_copy(sv_, acc.at[iv_], sem_sc, add=True)

        def drain(sv_, iv_):
            pltpu.make_async_copy(sv_, acc.at[iv_], sem_sc).wait()

        start_in(0, sv0, iv0)

        @pl.loop(0, _NPAGES // 2)
        def _(q):
            p0 = 2 * q

            @pl.when(q > 0)
            def _():
                drain(sv1, iv1)

            start_in(p0 + 1, sv1, iv1)
            wait_in(p0, sv0, iv0)
            fire(sv0, iv0)
            drain(sv0, iv0)

            @pl.when(q + 1 < _NPAGES // 2)
            def _():
                start_in(p0 + 2, sv0, iv0)

            wait_in(p0 + 1, sv1, iv1)
            fire(sv1, iv1)

        drain(sv1, iv1)

        @pl.when(wid < _XTILES)
        def _():
            e = _NW * _EPT + wid * _XELEM
            pltpu.sync_copy(src_hbm.at[pl.ds(e, _XELEM)], xv)
            pltpu.sync_copy(idx_hbm.at[pl.ds(e, _XELEM)], xi)
            pltpu.sync_copy(xv, acc.at[xi], add=True)

        plsc.subcore_barrier()

        @pl.when(sid == 0)
        def _():
            pltpu.sync_copy(acc, out_hbm.at[cid])

    return k(src1, batch1)


def _combine(partials):
    def body(p_ref, o_ref):
        o_ref[...] = p_ref[0, :] + p_ref[1, :]

    return pl.pallas_call(
        body, out_shape=jax.ShapeDtypeStruct((_G,), jnp.float32)
    )(partials)


@jax.jit
def _run(src, batch):
    return _combine(_sc_partials(src.reshape(_N), batch))


def kernel(src, batch, cell_volume):
    del cell_volume
    return _run(src, batch)


# speedup vs baseline: 36.4986x; 2.5197x over previous
import functools

import jax
import jax.numpy as jnp
from jax import lax
from jax.experimental import pallas as pl
from jax.experimental.pallas import tpu as pltpu
from jax.experimental.pallas import tpu_sc as plsc

_N = 6400000
_G = 4096
_NC = 2
_NS = 16
_NW = _NC * _NS
_BLK = 64
_GELEM = 16 * _BLK
_GPT = 195
_EPT = _GPT * _GELEM
_XTILES = (_N - _EPT * _NW) // _GELEM
_GPP = 15
_PAGE = _GPP * _GELEM
_NPAGES = _GPT // _GPP
_NENT = _GPT * 16
_RING = 8
_ZCHUNK = _G // _NS


def _sc_partials(src1, batch1):
    mesh = plsc.VectorSubcoreMesh(core_axis_name="c", subcore_axis_name="s")

    @functools.partial(
        pl.kernel,
        out_type=jax.ShapeDtypeStruct((_NC, _G), jnp.float32),
        mesh=mesh,
        compiler_params=pltpu.CompilerParams(needs_layout_passes=False),
        scratch_types=[
            pltpu.VMEM((_PAGE,), jnp.float32),
            pltpu.VMEM((_PAGE,), jnp.int32),
            pltpu.VMEM((_PAGE,), jnp.float32),
            pltpu.VMEM((_PAGE,), jnp.int32),
            pltpu.VMEM((_NENT,), jnp.float32),
            pltpu.VMEM((_NENT,), jnp.int32),
            pltpu.VMEM((_RING, _BLK), jnp.float32),
            pltpu.VMEM((_RING, _BLK), jnp.int32),
            pltpu.VMEM((_GELEM,), jnp.float32),
            pltpu.VMEM((_GELEM,), jnp.int32),
            pltpu.VMEM((_ZCHUNK,), jnp.float32),
            pltpu.VMEM_SHARED((_G,), jnp.float32),
            pltpu.SemaphoreType.DMA,
            pltpu.SemaphoreType.DMA,
        ],
    )
    def k(src_hbm, idx_hbm, out_hbm, sv0, iv0, sv1, iv1, bsl, bil,
          bbv, bbi, xv, xi, zv, acc, sem_in, sem_bb):
        cid = lax.axis_index("c")
        sid = lax.axis_index("s")
        wid = cid * _NS + sid

        @pl.loop(0, _ZCHUNK // 16, unroll=8)
        def _(i):
            zv[pl.ds(i * 16, 16)] = jnp.zeros((16,), jnp.float32)

        pltpu.sync_copy(zv, acc.at[pl.ds(sid * _ZCHUNK, _ZCHUNK)])
        plsc.subcore_barrier()

        base = wid * _EPT
        lanes = lax.iota(jnp.int32, 16)

        def start_in(p, sv_, iv_):
            e0 = base + p * _PAGE
            pltpu.async_copy(src_hbm.at[pl.ds(e0, _PAGE)], sv_, sem_in)
            pltpu.async_copy(idx_hbm.at[pl.ds(e0, _PAGE)], iv_, sem_in)

        def wait_in(p, sv_, iv_):
            e0 = base + p * _PAGE
            pltpu.make_async_copy(
                src_hbm.at[pl.ds(e0, _PAGE)], sv_, sem_in).wait()
            pltpu.make_async_copy(
                idx_hbm.at[pl.ds(e0, _PAGE)], iv_, sem_in).wait()

        def process_group(sv_, iv_, off, list_off, cnt):
            bidx = off + lanes * _BLK
            gfirst = plsc.load_gather(iv_, [bidx])
            glast = plsc.load_gather(iv_, [bidx + (_BLK - 1)])
            uniform = gfirst == glast

            a0 = plsc.load_gather(sv_, [bidx])
            a1 = plsc.load_gather(sv_, [bidx + 1])
            a2 = plsc.load_gather(sv_, [bidx + 2])
            a3 = plsc.load_gather(sv_, [bidx + 3])
            for i in range(4, _BLK, 4):
                a0 = a0 + plsc.load_gather(sv_, [bidx + i])
                a1 = a1 + plsc.load_gather(sv_, [bidx + i + 1])
                a2 = a2 + plsc.load_gather(sv_, [bidx + i + 2])
                a3 = a3 + plsc.load_gather(sv_, [bidx + i + 3])
            total = (a0 + a1) + (a2 + a3)

            bsl[pl.ds(list_off, 16)] = jnp.where(
                uniform, total, jnp.float32(0.0))
            bil[pl.ds(list_off, 16)] = gfirst

            nbi = jnp.logical_not(uniform).astype(jnp.int32)
            pre = plsc.cumsum(nbi)
            nfired = pre[15]

            @pl.when(nfired > 0)
            def _():
                excl = pre - nbi
                for j in range(16):
                    @pl.when(nbi[j] > 0)
                    def _():
                        fidx = cnt + excl[j]
                        slot = lax.rem(fidx, _RING)

                        @pl.when(fidx >= _RING)
                        def _():
                            pltpu.make_async_copy(
                                bbv.at[slot], acc.at[bbi.at[slot]],
                                sem_bb).wait()

                        o = off + j * _BLK
                        for t in range(_BLK // 16):
                            bbv[slot, pl.ds(16 * t, 16)] = (
                                sv_[pl.ds(o + 16 * t, 16)])
                            bbi[slot, pl.ds(16 * t, 16)] = (
                                iv_[pl.ds(o + 16 * t, 16)])
                        pltpu.async_copy(
                            bbv.at[slot], acc.at[bbi.at[slot]], sem_bb,
                            add=True)

            return cnt + nfired

        def process_page(p, sv_, iv_, cnt):
            @pl.loop(0, _GPP, init_carry=cnt)
            def gloop(g, cnt):
                return process_group(
                    sv_, iv_, g * _GELEM, (p * _GPP + g) * 16, cnt)

            return gloop

        start_in(0, sv0, iv0)

        @pl.loop(0, _NPAGES // 2, init_carry=jnp.int32(0))
        def pages(q, cnt):
            p0 = 2 * q
            start_in(p0 + 1, sv1, iv1)
            wait_in(p0, sv0, iv0)
            cnt = process_page(p0, sv0, iv0, cnt)

            @pl.when(q + 1 < _NPAGES // 2)
            def _():
                start_in(p0 + 2, sv0, iv0)

            @pl.when(q + 1 == _NPAGES // 2)
            def _():
                start_in(_NPAGES - 1, sv0, iv0)

            wait_in(p0 + 1, sv1, iv1)
            return process_page(p0 + 1, sv1, iv1, cnt)

        wait_in(_NPAGES - 1, sv0, iv0)
        cnt = process_page(_NPAGES - 1, sv0, iv0, pages)

        @pl.loop(0, _RING)
        def _(i):
            @pl.when(i < jnp.minimum(cnt, _RING))
            def _():
                pltpu.make_async_copy(
                    bbv.at[i], acc.at[bbi.at[i]], sem_bb).wait()

        pltpu.sync_copy(bsl, acc.at[bil], add=True)

        @pl.when(wid < _XTILES)
        def _():
            e = _NW * _EPT + wid * _GELEM
            pltpu.sync_copy(src_hbm.at[pl.ds(e, _GELEM)], xv)
            pltpu.sync_copy(idx_hbm.at[pl.ds(e, _GELEM)], xi)
            pltpu.sync_copy(xv, acc.at[xi], add=True)

        plsc.subcore_barrier()

        @pl.when(sid == 0)
        def _():
            pltpu.sync_copy(acc, out_hbm.at[cid])

    return k(src1, batch1)


def _combine(partials):
    def body(p_ref, o_ref):
        o_ref[...] = p_ref[0, :] + p_ref[1, :]

    return pl.pallas_call(
        body, out_shape=jax.ShapeDtypeStruct((_G,), jnp.float32)
    )(partials)


@jax.jit
def _run(src, batch):
    return _combine(_sc_partials(src.reshape(_N), batch))


def kernel(src, batch, cell_volume):
    del cell_volume
    return _run(src, batch)


# speedup vs baseline: 57.4265x; 1.5734x over previous
import functools

import jax
import jax.numpy as jnp
from jax import lax
from jax.experimental import pallas as pl
from jax.experimental.pallas import tpu as pltpu
from jax.experimental.pallas import tpu_sc as plsc

_N = 6400000
_G = 4096
_NC = 2
_NS = 16
_NW = _NC * _NS
_BLK = 64
_GELEM = 16 * _BLK
_GPT = 195
_EPT = _GPT * _GELEM
_XTILES = (_N - _EPT * _NW) // _GELEM
_GPP = 15
_PAGE = _GPP * _GELEM
_NPAGES = _GPT // _GPP
_NENT = _GPT * 16
_RING = 8
_ZCHUNK = _G // _NS


def _sc_partials(src1, batch1):
    mesh = plsc.VectorSubcoreMesh(core_axis_name="c", subcore_axis_name="s")

    @functools.partial(
        pl.kernel,
        out_type=jax.ShapeDtypeStruct((_NC, _G), jnp.float32),
        mesh=mesh,
        compiler_params=pltpu.CompilerParams(needs_layout_passes=False),
        scratch_types=[
            pltpu.VMEM((_PAGE,), jnp.float32),
            pltpu.VMEM((_PAGE,), jnp.int32),
            pltpu.VMEM((_PAGE,), jnp.float32),
            pltpu.VMEM((_PAGE,), jnp.int32),
            pltpu.VMEM((_NENT,), jnp.float32),
            pltpu.VMEM((_NENT,), jnp.int32),
            pltpu.VMEM((_RING, _BLK), jnp.float32),
            pltpu.VMEM((_RING, _BLK), jnp.int32),
            pltpu.VMEM((_GELEM,), jnp.float32),
            pltpu.VMEM((_GELEM,), jnp.int32),
            pltpu.VMEM((_ZCHUNK,), jnp.float32),
            pltpu.VMEM_SHARED((_G,), jnp.float32),
            pltpu.SemaphoreType.DMA,
            pltpu.SemaphoreType.DMA,
        ],
    )
    def k(src_hbm, idx_hbm, out_hbm, sv0, iv0, sv1, iv1, bsl, bil,
          bbv, bbi, xv, xi, zv, acc, sem_in, sem_bb):
        cid = lax.axis_index("c")
        sid = lax.axis_index("s")
        wid = cid * _NS + sid

        @pl.loop(0, _ZCHUNK // 16, unroll=8)
        def _(i):
            zv[pl.ds(i * 16, 16)] = jnp.zeros((16,), jnp.float32)

        pltpu.sync_copy(zv, acc.at[pl.ds(sid * _ZCHUNK, _ZCHUNK)])
        plsc.subcore_barrier()

        base = wid * _EPT
        lanes = lax.iota(jnp.int32, 16)

        def start_in(p, sv_, iv_):
            e0 = base + p * _PAGE
            pltpu.async_copy(src_hbm.at[pl.ds(e0, _PAGE)], sv_, sem_in)
            pltpu.async_copy(idx_hbm.at[pl.ds(e0, _PAGE)], iv_, sem_in)

        def wait_in(p, sv_, iv_):
            e0 = base + p * _PAGE
            pltpu.make_async_copy(
                src_hbm.at[pl.ds(e0, _PAGE)], sv_, sem_in).wait()
            pltpu.make_async_copy(
                idx_hbm.at[pl.ds(e0, _PAGE)], iv_, sem_in).wait()

        def process_group(sv_, iv_, off, list_off, cnt):
            bidx = off + lanes * _BLK
            gfirst = plsc.load_gather(iv_, [bidx])
            glast = plsc.load_gather(iv_, [bidx + (_BLK - 1)])
            uniform = gfirst == glast

            nacc = 8
            a = [plsc.load_gather(sv_, [bidx + i]) for i in range(nacc)]
            for i in range(nacc, _BLK, nacc):
                for t in range(nacc):
                    a[t] = a[t] + plsc.load_gather(sv_, [bidx + i + t])
            while len(a) > 1:
                a = [a[2 * t] + a[2 * t + 1] for t in range(len(a) // 2)]
            total = a[0]

            bsl[pl.ds(list_off, 16)] = jnp.where(
                uniform, total, jnp.float32(0.0))
            bil[pl.ds(list_off, 16)] = gfirst

            notuni = jnp.logical_not(uniform)
            nfired = plsc.all_reduce_population_count(notuni)
            nfired = nfired if jnp.ndim(nfired) == 0 else nfired[0]

            @pl.when(nfired > 0)
            def _():
                @pl.loop(0, nfired, init_carry=notuni.astype(jnp.int32))
                def _(f, m):
                    j = plsc.all_reduce_ffs(m != 0)
                    j = j if jnp.ndim(j) == 0 else j[0]
                    fidx = cnt + f
                    slot = lax.rem(fidx, _RING)

                    @pl.when(fidx >= _RING)
                    def _():
                        pltpu.make_async_copy(
                            bbv.at[slot], acc.at[bbi.at[slot]],
                            sem_bb).wait()

                    o = off + j * _BLK
                    for t in range(_BLK // 16):
                        bbv[slot, pl.ds(16 * t, 16)] = (
                            sv_[pl.ds(o + 16 * t, 16)])
                        bbi[slot, pl.ds(16 * t, 16)] = (
                            iv_[pl.ds(o + 16 * t, 16)])
                    pltpu.async_copy(
                        bbv.at[slot], acc.at[bbi.at[slot]], sem_bb,
                        add=True)
                    return m & (lanes != j).astype(jnp.int32)

            return cnt + nfired

        def process_page(p, sv_, iv_, cnt):
            @pl.loop(0, _GPP, init_carry=cnt)
            def gloop(g, cnt):
                return process_group(
                    sv_, iv_, g * _GELEM, (p * _GPP + g) * 16, cnt)

            return gloop

        start_in(0, sv0, iv0)

        @pl.loop(0, _NPAGES // 2, init_carry=jnp.int32(0))
        def pages(q, cnt):
            p0 = 2 * q
            start_in(p0 + 1, sv1, iv1)
            wait_in(p0, sv0, iv0)
            cnt = process_page(p0, sv0, iv0, cnt)

            @pl.when(q + 1 < _NPAGES // 2)
            def _():
                start_in(p0 + 2, sv0, iv0)

            @pl.when(q + 1 == _NPAGES // 2)
            def _():
                start_in(_NPAGES - 1, sv0, iv0)

            wait_in(p0 + 1, sv1, iv1)
            return process_page(p0 + 1, sv1, iv1, cnt)

        wait_in(_NPAGES - 1, sv0, iv0)
        cnt = process_page(_NPAGES - 1, sv0, iv0, pages)

        @pl.loop(0, _RING)
        def _(i):
            @pl.when(i < jnp.minimum(cnt, _RING))
            def _():
                pltpu.make_async_copy(
                    bbv.at[i], acc.at[bbi.at[i]], sem_bb).wait()

        pltpu.sync_copy(bsl, acc.at[bil], add=True)

        @pl.when(wid < _XTILES)
        def _():
            e = _NW * _EPT + wid * _GELEM
            pltpu.sync_copy(src_hbm.at[pl.ds(e, _GELEM)], xv)
            pltpu.sync_copy(idx_hbm.at[pl.ds(e, _GELEM)], xi)
            pltpu.sync_copy(xv, acc.at[xi], add=True)

        plsc.subcore_barrier()

        @pl.when(sid == 0)
        def _():
            pltpu.sync_copy(acc, out_hbm.at[cid])

    return k(src1, batch1)


def _combine(partials):
    def body(p_ref, o_ref):
        o_ref[...] = p_ref[0, :] + p_ref[1, :]

    return pl.pallas_call(
        body, out_shape=jax.ShapeDtypeStruct((_G,), jnp.float32)
    )(partials)


@jax.jit
def _run(src, batch):
    return _combine(_sc_partials(src.reshape(_N), batch))


def kernel(src, batch, cell_volume):
    del cell_volume
    return _run(src, batch)


# speedup vs baseline: 131.0984x; 2.2829x over previous
import functools

import jax
import jax.numpy as jnp
from jax import lax
from jax.experimental import pallas as pl
from jax.experimental.pallas import tpu as pltpu
from jax.experimental.pallas import tpu_sc as plsc

_N = 6400000
_G = 4096
_NC = 2
_NS = 16
_NW = _NC * _NS
_BLK = 63
_GELEM = 16 * _BLK
_GPT = 198
_EPT = _GPT * _GELEM
_XELEM = 1024
_XTILES = (_N - _EPT * _NW) // _XELEM
_GPP = 11
_PAGE = _GPP * _GELEM
_NPAGES = _GPT // _GPP
_NENT = _GPT * 16
_RING = 8
_ZCHUNK = _G // _NS


def _sc_partials(src1, batch1):
    mesh = plsc.VectorSubcoreMesh(core_axis_name="c", subcore_axis_name="s")

    @functools.partial(
        pl.kernel,
        out_type=jax.ShapeDtypeStruct((_NC, _G), jnp.float32),
        mesh=mesh,
        compiler_params=pltpu.CompilerParams(needs_layout_passes=False),
        scratch_types=[
            pltpu.VMEM((_PAGE + 16,), jnp.float32),
            pltpu.VMEM((_PAGE + 16,), jnp.int32),
            pltpu.VMEM((_PAGE + 16,), jnp.float32),
            pltpu.VMEM((_PAGE + 16,), jnp.int32),
            pltpu.VMEM((_NENT,), jnp.float32),
            pltpu.VMEM((_NENT,), jnp.int32),
            pltpu.VMEM((_RING, 64), jnp.float32),
            pltpu.VMEM((_RING, 64), jnp.int32),
            pltpu.VMEM((_XELEM,), jnp.float32),
            pltpu.VMEM((_XELEM,), jnp.int32),
            pltpu.VMEM((_ZCHUNK,), jnp.float32),
            pltpu.VMEM_SHARED((_G,), jnp.float32),
            pltpu.SemaphoreType.DMA,
            pltpu.SemaphoreType.DMA,
        ],
    )
    def k(src_hbm, idx_hbm, out_hbm, sv0, iv0, sv1, iv1, bsl, bil,
          bbv, bbi, xv, xi, zv, acc, sem_in, sem_bb):
        cid = lax.axis_index("c")
        sid = lax.axis_index("s")
        wid = cid * _NS + sid

        @pl.loop(0, _ZCHUNK // 16, unroll=8)
        def _(i):
            zv[pl.ds(i * 16, 16)] = jnp.zeros((16,), jnp.float32)

        pltpu.sync_copy(zv, acc.at[pl.ds(sid * _ZCHUNK, _ZCHUNK)])
        plsc.subcore_barrier()

        base = wid * _EPT
        lanes = lax.iota(jnp.int32, 16)
        last_lane = lanes == 15

        def start_in(p, sv_, iv_):
            e0 = base + p * _PAGE
            pltpu.async_copy(src_hbm.at[pl.ds(e0, _PAGE)],
                             sv_.at[pl.ds(0, _PAGE)], sem_in)
            pltpu.async_copy(idx_hbm.at[pl.ds(e0, _PAGE)],
                             iv_.at[pl.ds(0, _PAGE)], sem_in)

        def wait_in(p, sv_, iv_):
            e0 = base + p * _PAGE
            pltpu.make_async_copy(
                src_hbm.at[pl.ds(e0, _PAGE)],
                sv_.at[pl.ds(0, _PAGE)], sem_in).wait()
            pltpu.make_async_copy(
                idx_hbm.at[pl.ds(e0, _PAGE)],
                iv_.at[pl.ds(0, _PAGE)], sem_in).wait()

        def process_group(sv_, iv_, off, list_off, cnt):
            bidx = off + lanes * _BLK
            gfirst = plsc.load_gather(iv_, [bidx])
            glast = plsc.load_gather(iv_, [bidx + (_BLK - 1)])
            uniform = gfirst == glast

            nacc = 8
            a = [plsc.load_gather(sv_, [bidx + i]) for i in range(nacc)]
            for i in range(nacc, _BLK - (_BLK % nacc), nacc):
                for t in range(nacc):
                    a[t] = a[t] + plsc.load_gather(sv_, [bidx + i + t])
            for t in range(_BLK % nacc):
                a[t] = a[t] + plsc.load_gather(
                    sv_, [bidx + (_BLK - (_BLK % nacc)) + t])
            while len(a) > 1:
                a = [a[2 * t] + a[2 * t + 1] for t in range(len(a) // 2)]
            total = a[0]

            bsl[pl.ds(list_off, 16)] = jnp.where(
                uniform, total, jnp.float32(0.0))
            bil[pl.ds(list_off, 16)] = gfirst

            notuni = jnp.logical_not(uniform)
            nfired = plsc.all_reduce_population_count(notuni)
            nfired = nfired if jnp.ndim(nfired) == 0 else nfired[0]

            @pl.when(nfired > 0)
            def _():
                @pl.loop(0, nfired, init_carry=notuni.astype(jnp.int32))
                def _(f, m):
                    j = plsc.all_reduce_ffs(m != 0)
                    j = j if jnp.ndim(j) == 0 else j[0]
                    fidx = cnt + f
                    slot = lax.rem(fidx, _RING)

                    @pl.when(fidx >= _RING)
                    def _():
                        pltpu.make_async_copy(
                            bbv.at[slot], acc.at[bbi.at[slot]],
                            sem_bb).wait()

                    o = off + j * _BLK
                    for t in range(3):
                        bbv[slot, pl.ds(16 * t, 16)] = (
                            sv_[pl.ds(o + 16 * t, 16)])
                        bbi[slot, pl.ds(16 * t, 16)] = (
                            iv_[pl.ds(o + 16 * t, 16)])
                    bbv[slot, pl.ds(48, 16)] = jnp.where(
                        last_lane, jnp.float32(0.0), sv_[pl.ds(o + 48, 16)])
                    bbi[slot, pl.ds(48, 16)] = jnp.where(
                        last_lane, 0, iv_[pl.ds(o + 48, 16)])
                    pltpu.async_copy(
                        bbv.at[slot], acc.at[bbi.at[slot]], sem_bb,
                        add=True)
                    return m & (lanes != j).astype(jnp.int32)

            return cnt + nfired

        def process_page(p, sv_, iv_, cnt):
            @pl.loop(0, _GPP, init_carry=cnt)
            def gloop(g, cnt):
                return process_group(
                    sv_, iv_, g * _GELEM, (p * _GPP + g) * 16, cnt)

            return gloop

        start_in(0, sv0, iv0)

        @pl.loop(0, _NPAGES // 2, init_carry=jnp.int32(0))
        def pages(q, cnt):
            p0 = 2 * q
            start_in(p0 + 1, sv1, iv1)
            wait_in(p0, sv0, iv0)
            cnt = process_page(p0, sv0, iv0, cnt)

            @pl.when(q + 1 < _NPAGES // 2)
            def _():
                start_in(p0 + 2, sv0, iv0)

            wait_in(p0 + 1, sv1, iv1)
            return process_page(p0 + 1, sv1, iv1, cnt)

        cnt = pages

        @pl.loop(0, _RING)
        def _(i):
            @pl.when(i < jnp.minimum(cnt, _RING))
            def _():
                pltpu.make_async_copy(
                    bbv.at[i], acc.at[bbi.at[i]], sem_bb).wait()

        pltpu.sync_copy(bsl, acc.at[bil], add=True)

        @pl.when(wid < _XTILES)
        def _():
            e = _NW * _EPT + wid * _XELEM
            pltpu.sync_copy(src_hbm.at[pl.ds(e, _XELEM)], xv)
            pltpu.sync_copy(idx_hbm.at[pl.ds(e, _XELEM)], xi)
            pltpu.sync_copy(xv, acc.at[xi], add=True)

        plsc.subcore_barrier()

        @pl.when(sid == 0)
        def _():
            pltpu.sync_copy(acc, out_hbm.at[cid])

    return k(src1, batch1)


def _combine(partials):
    def body(p_ref, o_ref):
        o_ref[...] = p_ref[0, :] + p_ref[1, :]

    return pl.pallas_call(
        body, out_shape=jax.ShapeDtypeStruct((_G,), jnp.float32)
    )(partials)


@jax.jit
def _run(src, batch):
    return _combine(_sc_partials(src.reshape(_N), batch))


def kernel(src, batch, cell_volume):
    del cell_volume
    return _run(src, batch)
